# Initial kernel scaffold; baseline (speedup 1.0000x reference)
import jax
import jax.numpy as jnp
from jax.experimental import pallas as pl


def kernel(input_ids, table):
    raise NotImplementedError("write your pallas kernel here")


# speedup vs baseline: 1.8611x; 1.8611x over previous
import functools

import jax
import jax.numpy as jnp
from jax import lax
from jax.experimental import pallas as pl
from jax.experimental.pallas import tpu as pltpu
from jax.experimental.pallas import tpu_sc as plsc

CHUNK = 512


def _emb_kernel(n_per_w, n_chunks, num_cores, idx_hbm, table_hbm, out_hbm,
                idx_v, r0, r1, sg0, sg1, ss0, ss1):
    wid = lax.axis_index("s") * num_cores + lax.axis_index("c")
    base = wid * n_per_w

    pltpu.sync_copy(idx_hbm.at[pl.ds(base, n_per_w)], idx_v)

    def start_gather(c, r, sem):
        pltpu.async_copy(table_hbm.at[idx_v.at[pl.ds(c * CHUNK, CHUNK)]],
                         r, sem)

    def wait_gather(r, sem):
        pltpu.make_async_copy(
            table_hbm.at[idx_v.at[pl.ds(0, CHUNK)]], r, sem).wait()

    def start_store(c, r, sem):
        pltpu.async_copy(r, out_hbm.at[pl.ds(base + c * CHUNK, CHUNK)], sem)

    def wait_store(r, sem):
        pltpu.make_async_copy(r, out_hbm.at[pl.ds(base, CHUNK)], sem).wait()

    last = n_chunks - 1
    start_gather(0, r0, sg0)
    start_gather(1, r1, sg1)

    def body(g, carry):
        a = 2 * g
        wait_gather(r0, sg0)
        start_store(a, r0, ss0)
        wait_gather(r1, sg1)
        start_store(a + 1, r1, ss1)
        c0 = jnp.minimum(a + 2, last)
        c1 = jnp.minimum(a + 3, last)
        wait_store(r0, ss0)
        start_gather(c0, r0, sg0)
        wait_store(r1, ss1)
        start_gather(c1, r1, sg1)
        return carry

    lax.fori_loop(0, n_chunks // 2, body, 0)
    wait_gather(r0, sg0)
    wait_gather(r1, sg1)


def kernel(input_ids, table):
    B, S = input_ids.shape
    V, D = table.shape
    N = B * S
    idx = input_ids.reshape(N).astype(jnp.int32)

    info = plsc.get_sparse_core_info()
    nw = info.num_cores * info.num_subcores
    n_per_w = N // nw
    assert n_per_w * nw == N and n_per_w % (2 * CHUNK) == 0
    n_chunks = n_per_w // CHUNK

    mesh = plsc.VectorSubcoreMesh(core_axis_name="c", subcore_axis_name="s")
    emb = functools.partial(
        pl.kernel,
        mesh=mesh,
        out_type=jax.ShapeDtypeStruct((N, D), jnp.float32),
        scratch_types=[
            pltpu.VMEM((n_per_w,), jnp.int32),
            pltpu.VMEM((CHUNK, D), jnp.float32),
            pltpu.VMEM((CHUNK, D), jnp.float32),
            pltpu.SemaphoreType.DMA,
            pltpu.SemaphoreType.DMA,
            pltpu.SemaphoreType.DMA,
            pltpu.SemaphoreType.DMA,
        ],
        compiler_params=pltpu.CompilerParams(use_tc_tiling_on_sc=False),
    )(functools.partial(_emb_kernel, n_per_w, n_chunks, info.num_cores))

    out = emb(idx, table)
    return out.reshape(B, S, D)
